# Initial kernel scaffold; baseline (speedup 1.0000x reference)
import jax
import jax.numpy as jnp
from jax.experimental import pallas as pl


def kernel(merchant_id, category, mcc, country, currency, hour_of_day, emb_merchant_id, emb_category, emb_mcc, emb_country, emb_currency, emb_hour_of_day, W, b):
    raise NotImplementedError("write your pallas kernel here")


# speedup vs baseline: 7.4266x; 7.4266x over previous
import functools

import jax
import jax.numpy as jnp
from jax import lax
from jax.experimental import pallas as pl
from jax.experimental.pallas import tpu as pltpu
from jax.experimental.pallas import tpu_sc as plsc

B, L = 4096, 200
N = B * L
DIMS = (32, 16, 16, 16, 16, 16)
TOTAL = sum(DIMS)
PROJ = 128

NC, NS = 2, 16
NW = NC * NS
PER_W = N // NW
CHUNK = 512
KROW = CHUNK // 128
STEPS = PER_W // CHUNK
IDX_ROWS = N // 128


def _sc_gather(idxs, tabs):
    mesh = plsc.VectorSubcoreMesh(core_axis_name="c", subcore_axis_name="s")
    out_type = tuple(jax.ShapeDtypeStruct((N, d), jnp.float32) for d in DIMS)
    scratch = []
    for d in DIMS:
        scratch.append(pltpu.VMEM((KROW, 128), jnp.int32))
        scratch.append(pltpu.VMEM((CHUNK, d), jnp.float32))
    scratch.append(pltpu.SemaphoreType.DMA)

    @functools.partial(pl.kernel, out_type=out_type, mesh=mesh,
                       scratch_types=scratch,
                       compiler_params=pltpu.CompilerParams(
                           use_tc_tiling_on_sc=False))
    def body(i0, i1, i2, i3, i4, i5, t0, t1, t2, t3, t4, t5,
             o0, o1, o2, o3, o4, o5,
             xi0, xr0, xi1, xr1, xi2, xr2, xi3, xr3, xi4, xr4, xi5, xr5,
             sem):
        idx_refs = (i0, i1, i2, i3, i4, i5)
        tab_refs = (t0, t1, t2, t3, t4, t5)
        out_refs = (o0, o1, o2, o3, o4, o5)
        ibufs = (xi0, xi1, xi2, xi3, xi4, xi5)
        rbufs = (xr0, xr1, xr2, xr3, xr4, xr5)
        wid = lax.axis_index("s") * NC + lax.axis_index("c")

        def step(g, carry):
            base = (wid * STEPS + g) * CHUNK
            row0 = (wid * STEPS + g) * KROW
            for f in range(6):
                pltpu.sync_copy(idx_refs[f].at[pl.ds(row0, KROW)], ibufs[f])
            descs = []
            for f in range(6):
                for j in range(KROW):
                    descs.append(pltpu.async_copy(
                        tab_refs[f].at[ibufs[f].at[j]],
                        rbufs[f].at[pl.ds(j * 128, 128)], sem))
            for d in descs:
                d.wait()
            for f in range(6):
                pltpu.sync_copy(rbufs[f], out_refs[f].at[pl.ds(base, CHUNK)])
            return carry

        lax.fori_loop(0, STEPS, step, 0)

    return body(*idxs, *tabs)


def _tc_project(gs, w_t, b2):
    BLK = 2048
    grid = (N // BLK,)
    in_specs = [pl.BlockSpec((BLK, d), lambda i: (i, 0)) for d in DIMS]
    in_specs.append(pl.BlockSpec((TOTAL, PROJ), lambda i: (0, 0)))
    in_specs.append(pl.BlockSpec((1, PROJ), lambda i: (0, 0)))

    def body(g0, g1, g2, g3, g4, g5, wt, bb, out):
        acc = jnp.dot(g0[...], wt[0:32, :], preferred_element_type=jnp.float32)
        off = 32
        for gref in (g1, g2, g3, g4, g5):
            acc = acc + jnp.dot(gref[...], wt[off:off + 16, :],
                                preferred_element_type=jnp.float32)
            off += 16
        out[...] = acc + bb[...]

    return pl.pallas_call(
        body, grid=grid, in_specs=in_specs,
        out_specs=pl.BlockSpec((BLK, PROJ), lambda i: (i, 0)),
        out_shape=jax.ShapeDtypeStruct((N, PROJ), jnp.float32),
    )(*gs, w_t, b2)


def kernel(merchant_id, category, mcc, country, currency, hour_of_day,
           emb_merchant_id, emb_category, emb_mcc, emb_country, emb_currency,
           emb_hour_of_day, W, b):
    idxs = [a.reshape(IDX_ROWS, 128) for a in
            (merchant_id, category, mcc, country, currency, hour_of_day)]
    tabs = (emb_merchant_id, emb_category, emb_mcc, emb_country, emb_currency,
            emb_hour_of_day)
    gs = _sc_gather(idxs, tabs)
    out = _tc_project(gs, W.T, b.reshape(1, PROJ))
    return out.reshape(B, L, PROJ)


# speedup vs baseline: 9.9910x; 1.3453x over previous
import functools

import jax
import jax.numpy as jnp
from jax import lax
from jax.experimental import pallas as pl
from jax.experimental.pallas import tpu as pltpu
from jax.experimental.pallas import tpu_sc as plsc

B, L = 4096, 200
N = B * L
DIMS = (32, 16, 16, 16, 16, 32)
OFFS = (0, 32, 48, 64, 80, 96)
TOTAL = 112
PROJ = 128

NC, NS = 2, 16
NW = NC * NS
PER_W = N // NW
CHUNK = 256
KROW = CHUNK // 128
STEPS = PER_W // CHUNK
IDX_ROWS = N // 128


def _sc_gather_concat(idxs, tabs):
    mesh = plsc.VectorSubcoreMesh(core_axis_name="c", subcore_axis_name="s")
    out_type = jax.ShapeDtypeStruct((N, PROJ), jnp.float32)
    scratch = []
    for _b in range(2):
        for _f in range(6):
            scratch.append(pltpu.VMEM((KROW, 128), jnp.int32))
        for d in DIMS:
            scratch.append(pltpu.VMEM((CHUNK, d), jnp.float32))
    for _s in range(5):
        scratch.append(pltpu.SemaphoreType.DMA)

    @functools.partial(pl.kernel, out_type=out_type, mesh=mesh,
                       scratch_types=scratch,
                       compiler_params=pltpu.CompilerParams(
                           use_tc_tiling_on_sc=False))
    def body(i0, i1, i2, i3, i4, i5, t0, t1, t2, t3, t4, t5, out,
             a0, a1, a2, a3, a4, a5, ra0, ra1, ra2, ra3, ra4, ra5,
             b0, b1, b2, b3, b4, b5, rb0, rb1, rb2, rb3, rb4, rb5,
             semi0, semi1, semg, semw0, semw1):
        idx_refs = (i0, i1, i2, i3, i4, i5)
        tab_refs = (t0, t1, t2, t3, t4, t5)
        ibufs = ((a0, a1, a2, a3, a4, a5), (b0, b1, b2, b3, b4, b5))
        rbufs = ((ra0, ra1, ra2, ra3, ra4, ra5),
                 (rb0, rb1, rb2, rb3, rb4, rb5))
        semi = (semi0, semi1)
        semw = (semw0, semw1)
        wid = lax.axis_index("s") * NC + lax.axis_index("c")
        row_base = wid * (PER_W // 128)
        pos_base = wid * PER_W

        def fire_idx(g, b):
            for f in range(6):
                pltpu.async_copy(
                    idx_refs[f].at[pl.ds(row_base + g * KROW, KROW)],
                    ibufs[b][f], semi[b])

        def drain_idx(b):
            for f in range(6):
                pltpu.make_async_copy(
                    idx_refs[f].at[pl.ds(row_base, KROW)],
                    ibufs[b][f], semi[b]).wait()

        def fire_write(g, b):
            for f in range(6):
                pltpu.async_copy(
                    rbufs[b][f],
                    out.at[pl.ds(pos_base + g * CHUNK, CHUNK),
                           pl.ds(OFFS[f], DIMS[f])], semw[b])

        def drain_write(b):
            for f in range(6):
                pltpu.make_async_copy(
                    rbufs[b][f],
                    out.at[pl.ds(pos_base, CHUNK),
                           pl.ds(OFFS[f], DIMS[f])], semw[b]).wait()

        def do_chunk(g, b, first):
            if not first:
                drain_write(b)
            drain_idx(b)
            descs = []
            for f in range(6):
                for j in range(KROW):
                    descs.append(pltpu.async_copy(
                        tab_refs[f].at[ibufs[b][f].at[j]],
                        rbufs[b][f].at[pl.ds(j * 128, 128)], semg))
            nxt = jnp.minimum(g + 1, STEPS - 1)
            fire_idx(nxt, 1 - b)
            for d in descs:
                d.wait()
            fire_write(g, b)

        fire_idx(jnp.int32(0), 0)
        do_chunk(jnp.int32(0), 0, True)
        do_chunk(jnp.int32(1), 1, True)

        def step(g2, carry):
            do_chunk(g2 * 2, 0, False)
            do_chunk(g2 * 2 + 1, 1, False)
            return carry

        lax.fori_loop(1, STEPS // 2, step, 0)

        drain_write(0)
        drain_write(1)
        drain_idx(0)

    return body(*idxs, *tabs)


def _tc_project(cat, w_t_pad, b2):
    BLK = 4096
    grid = (N // BLK,)

    def body(x, wt, bb, out):
        out[...] = jnp.dot(x[...], wt[...],
                           preferred_element_type=jnp.float32) + bb[...]

    return pl.pallas_call(
        body, grid=grid,
        in_specs=[pl.BlockSpec((BLK, PROJ), lambda i: (i, 0)),
                  pl.BlockSpec((PROJ, PROJ), lambda i: (0, 0)),
                  pl.BlockSpec((1, PROJ), lambda i: (0, 0))],
        out_specs=pl.BlockSpec((BLK, PROJ), lambda i: (i, 0)),
        out_shape=jax.ShapeDtypeStruct((N, PROJ), jnp.float32),
    )(cat, w_t_pad, b2)


def kernel(merchant_id, category, mcc, country, currency, hour_of_day,
           emb_merchant_id, emb_category, emb_mcc, emb_country, emb_currency,
           emb_hour_of_day, W, b):
    idxs = [a.reshape(IDX_ROWS, 128) for a in
            (merchant_id, category, mcc, country, currency, hour_of_day)]
    hour_pad = jnp.pad(emb_hour_of_day, ((0, 0), (0, 16)))
    tabs = (emb_merchant_id, emb_category, emb_mcc, emb_country, emb_currency,
            hour_pad)
    cat = _sc_gather_concat(idxs, tabs)
    w_t_pad = jnp.zeros((PROJ, PROJ), jnp.float32).at[:TOTAL].set(W.T)
    out = _tc_project(cat, w_t_pad, b.reshape(1, PROJ))
    return out.reshape(B, L, PROJ)


# speedup vs baseline: 11.3634x; 1.1374x over previous
import functools

import jax
import jax.numpy as jnp
from jax import lax
from jax.experimental import pallas as pl
from jax.experimental.pallas import tpu as pltpu
from jax.experimental.pallas import tpu_sc as plsc

B, L = 4096, 200
N = B * L
DIMS = (32, 16, 16, 16, 16, 32)
TOTAL = 112
PROJ = 128

NC, NS = 2, 16
NW = NC * NS
PER_W = N // NW
CHUNK = 256
KROW = CHUNK // 128
STEPS = PER_W // CHUNK
IDX_ROWS = N // 128

_PACK = ((0, 0, 0), (0, 16, 16), (1, 0, 32), (2, 0, 48), (3, 0, 64),
         (4, 0, 80), (5, 0, 96), (5, 16, 112))


def _sc_gather_concat(idxs, tabs):
    mesh = plsc.VectorSubcoreMesh(core_axis_name="c", subcore_axis_name="s")
    out_type = jax.ShapeDtypeStruct((N, PROJ), jnp.float32)
    scratch = []
    for _b in range(2):
        for _f in range(6):
            scratch.append(pltpu.VMEM((KROW, 128), jnp.int32))
        for d in DIMS:
            scratch.append(pltpu.VMEM((CHUNK, d), jnp.float32))
    scratch.append(pltpu.VMEM((CHUNK, PROJ), jnp.float32))
    for _s in range(5):
        scratch.append(pltpu.SemaphoreType.DMA)

    @functools.partial(pl.kernel, out_type=out_type, mesh=mesh,
                       scratch_types=scratch,
                       compiler_params=pltpu.CompilerParams(
                           use_tc_tiling_on_sc=False))
    def body(i0, i1, i2, i3, i4, i5, t0, t1, t2, t3, t4, t5, out,
             a0, a1, a2, a3, a4, a5, ra0, ra1, ra2, ra3, ra4, ra5,
             b0, b1, b2, b3, b4, b5, rb0, rb1, rb2, rb3, rb4, rb5,
             cat, semi0, semi1, semg0, semg1, semw):
        idx_refs = (i0, i1, i2, i3, i4, i5)
        tab_refs = (t0, t1, t2, t3, t4, t5)
        ibufs = ((a0, a1, a2, a3, a4, a5), (b0, b1, b2, b3, b4, b5))
        rbufs = ((ra0, ra1, ra2, ra3, ra4, ra5),
                 (rb0, rb1, rb2, rb3, rb4, rb5))
        semi = (semi0, semi1)
        semg = (semg0, semg1)
        wid = lax.axis_index("s") * NC + lax.axis_index("c")
        row_base = wid * (PER_W // 128)
        pos_base = wid * PER_W

        def fire_idx(g, sl):
            g = jnp.minimum(g, STEPS - 1)
            for f in range(6):
                pltpu.async_copy(
                    idx_refs[f].at[pl.ds(row_base + g * KROW, KROW)],
                    ibufs[sl][f], semi[sl])

        def drain_idx(sl):
            for f in range(6):
                pltpu.make_async_copy(
                    idx_refs[f].at[pl.ds(row_base, KROW)],
                    ibufs[sl][f], semi[sl]).wait()

        def fire_gathers(sl):
            for f in range(6):
                for j in range(KROW):
                    pltpu.async_copy(
                        tab_refs[f].at[ibufs[sl][f].at[j]],
                        rbufs[sl][f].at[pl.ds(j * 128, 128)], semg[sl])

        def wait_gathers(sl):
            for f in range(6):
                for j in range(KROW):
                    pltpu.make_async_copy(
                        tab_refs[f].at[ibufs[sl][f].at[j]],
                        rbufs[sl][f].at[pl.ds(j * 128, 128)],
                        semg[sl]).wait()

        def repack(sl):
            def row(r, carry):
                for f, soff, doff in _PACK:
                    cat[r, pl.ds(doff, 16)] = rbufs[sl][f][r, pl.ds(soff, 16)]
                return carry
            lax.fori_loop(0, CHUNK, row, 0)

        def fire_write(g):
            pltpu.async_copy(
                cat, out.at[pl.ds(pos_base + g * CHUNK, CHUNK)], semw)

        def drain_write():
            pltpu.make_async_copy(
                cat, out.at[pl.ds(pos_base, CHUNK)], semw).wait()

        def half(c, sl, fire_next, first_w):
            other = 1 - sl
            if fire_next:
                drain_idx(other)
                fire_gathers(other)
            wait_gathers(sl)
            fire_idx(c + 2, sl)
            if not first_w:
                drain_write()
            repack(sl)
            fire_write(c)

        fire_idx(jnp.int32(0), 0)
        fire_idx(jnp.int32(1), 1)
        drain_idx(0)
        fire_gathers(0)
        half(jnp.int32(0), 0, True, True)
        half(jnp.int32(1), 1, True, False)

        def step(k, carry):
            half(2 * k, 0, True, False)
            half(2 * k + 1, 1, True, False)
            return carry

        lax.fori_loop(1, STEPS // 2 - 1, step, 0)

        half(jnp.int32(STEPS - 2), 0, True, False)
        half(jnp.int32(STEPS - 1), 1, False, False)

        drain_write()
        drain_idx(0)
        drain_idx(1)

    return body(*idxs, *tabs)


def _tc_project(cat, w_t_pad, b2):
    BLK = 4096
    grid = (N // BLK,)

    def body(x, wt, bb, out):
        out[...] = jnp.dot(x[...], wt[...],
                           preferred_element_type=jnp.float32) + bb[...]

    return pl.pallas_call(
        body, grid=grid,
        in_specs=[pl.BlockSpec((BLK, PROJ), lambda i: (i, 0)),
                  pl.BlockSpec((PROJ, PROJ), lambda i: (0, 0)),
                  pl.BlockSpec((1, PROJ), lambda i: (0, 0))],
        out_specs=pl.BlockSpec((BLK, PROJ), lambda i: (i, 0)),
        out_shape=jax.ShapeDtypeStruct((N, PROJ), jnp.float32),
    )(cat, w_t_pad, b2)


def kernel(merchant_id, category, mcc, country, currency, hour_of_day,
           emb_merchant_id, emb_category, emb_mcc, emb_country, emb_currency,
           emb_hour_of_day, W, b):
    idxs = [a.reshape(IDX_ROWS, 128) for a in
            (merchant_id, category, mcc, country, currency, hour_of_day)]
    hour_pad = jnp.pad(emb_hour_of_day, ((0, 0), (0, 16)))
    tabs = (emb_merchant_id, emb_category, emb_mcc, emb_country, emb_currency,
            hour_pad)
    cat = _sc_gather_concat(idxs, tabs)
    w_t_pad = jnp.zeros((PROJ, PROJ), jnp.float32).at[:TOTAL].set(W.T)
    out = _tc_project(cat, w_t_pad, b.reshape(1, PROJ))
    return out.reshape(B, L, PROJ)


# speedup vs baseline: 20.8053x; 1.8309x over previous
import functools

import jax
import jax.numpy as jnp
from jax import lax
from jax.experimental import pallas as pl
from jax.experimental.pallas import tpu as pltpu
from jax.experimental.pallas import tpu_sc as plsc

B, L = 4096, 200
N = B * L
DIMS = (32, 16, 16, 16, 16, 32)
TOTAL = 112
PROJ = 128

NC, NS = 2, 16
NW = NC * NS
PER_W = N // NW
CHUNK = 256
KROW = CHUNK // 128
STEPS = PER_W // CHUNK
IDX_ROWS = N // 128

_PACK = ((0, 0, 0), (0, 16, 16), (1, 0, 32), (2, 0, 48), (3, 0, 64),
         (4, 0, 80), (5, 0, 96), (5, 16, 112))


def _sc_gather_concat(idxs, tabs):
    mesh = plsc.VectorSubcoreMesh(core_axis_name="c", subcore_axis_name="s")
    out_type = jax.ShapeDtypeStruct((N, PROJ), jnp.float32)
    scratch = []
    for _b in range(2):
        for _f in range(6):
            scratch.append(pltpu.VMEM((KROW, 128), jnp.int32))
        for d in DIMS:
            scratch.append(pltpu.VMEM((CHUNK, d), jnp.float32))
    scratch.append(pltpu.VMEM((CHUNK, PROJ), jnp.float32))
    for _s in range(5):
        scratch.append(pltpu.SemaphoreType.DMA)

    @functools.partial(pl.kernel, out_type=out_type, mesh=mesh,
                       scratch_types=scratch,
                       compiler_params=pltpu.CompilerParams(
                           use_tc_tiling_on_sc=False))
    def body(i0, i1, i2, i3, i4, i5, t0, t1, t2, t3, t4, t5, out,
             a0, a1, a2, a3, a4, a5, ra0, ra1, ra2, ra3, ra4, ra5,
             b0, b1, b2, b3, b4, b5, rb0, rb1, rb2, rb3, rb4, rb5,
             cat, semi0, semi1, semg0, semg1, semw):
        idx_refs = (i0, i1, i2, i3, i4, i5)
        tab_refs = (t0, t1, t2, t3, t4, t5)
        ibufs = ((a0, a1, a2, a3, a4, a5), (b0, b1, b2, b3, b4, b5))
        rbufs = ((ra0, ra1, ra2, ra3, ra4, ra5),
                 (rb0, rb1, rb2, rb3, rb4, rb5))
        semi = (semi0, semi1)
        semg = (semg0, semg1)
        wid = lax.axis_index("s") * NC + lax.axis_index("c")
        row_base = wid * (PER_W // 128)
        pos_base = wid * PER_W

        def fire_idx(g, sl):
            g = jnp.minimum(g, STEPS - 1)
            for f in range(6):
                pltpu.async_copy(
                    idx_refs[f].at[pl.ds(row_base + g * KROW, KROW)],
                    ibufs[sl][f], semi[sl])

        def drain_idx(sl):
            for f in range(6):
                pltpu.make_async_copy(
                    idx_refs[f].at[pl.ds(row_base, KROW)],
                    ibufs[sl][f], semi[sl]).wait()

        def fire_gathers(sl):
            for f in range(6):
                for j in range(KROW):
                    pltpu.async_copy(
                        tab_refs[f].at[ibufs[sl][f].at[j]],
                        rbufs[sl][f].at[pl.ds(j * 128, 128)], semg[sl])

        def wait_gathers(sl):
            for f in range(6):
                for j in range(KROW):
                    pltpu.make_async_copy(
                        tab_refs[f].at[ibufs[sl][f].at[j]],
                        rbufs[sl][f].at[pl.ds(j * 128, 128)],
                        semg[sl]).wait()

        def repack(sl):
            def row(r, carry):
                for f, soff, doff in _PACK:
                    cat[r, pl.ds(doff, 16)] = rbufs[sl][f][r, pl.ds(soff, 16)]
                return carry
            lax.fori_loop(0, CHUNK, row, 0)

        def fire_write(g):
            pltpu.async_copy(
                cat, out.at[pl.ds(pos_base + g * CHUNK, CHUNK)], semw)

        def drain_write():
            pltpu.make_async_copy(
                cat, out.at[pl.ds(pos_base, CHUNK)], semw).wait()

        def half(c, sl, fire_next, first_w):
            other = 1 - sl
            if fire_next:
                drain_idx(other)
                fire_gathers(other)
            wait_gathers(sl)
            fire_idx(c + 2, sl)
            if not first_w:
                drain_write()
            repack(sl)
            fire_write(c)

        fire_idx(jnp.int32(0), 0)
        fire_idx(jnp.int32(1), 1)
        drain_idx(0)
        fire_gathers(0)
        half(jnp.int32(0), 0, True, True)
        half(jnp.int32(1), 1, True, False)

        def step(k, carry):
            half(2 * k, 0, True, False)
            half(2 * k + 1, 1, True, False)
            return carry

        lax.fori_loop(1, STEPS // 2 - 1, step, 0)

        half(jnp.int32(STEPS - 2), 0, True, False)
        half(jnp.int32(STEPS - 1), 1, False, False)

        drain_write()
        drain_idx(0)
        drain_idx(1)

    return body(*idxs, *tabs)


def _tc_project(cat, w_t_pad, b2):
    BLK = 4096
    grid = (N // BLK,)

    def body(x, wt, bb, out):
        out[...] = jnp.dot(x[...], wt[...],
                           preferred_element_type=jnp.float32) + bb[...]

    return pl.pallas_call(
        body, grid=grid,
        in_specs=[pl.BlockSpec((BLK, PROJ), lambda i: (i, 0)),
                  pl.BlockSpec((PROJ, PROJ), lambda i: (0, 0)),
                  pl.BlockSpec((1, PROJ), lambda i: (0, 0))],
        out_specs=pl.BlockSpec((BLK, PROJ), lambda i: (i, 0)),
        out_shape=jax.ShapeDtypeStruct((N, PROJ), jnp.float32),
    )(cat, w_t_pad, b2)


REP = 128


def kernel(merchant_id, category, mcc, country, currency, hour_of_day,
           emb_merchant_id, emb_category, emb_mcc, emb_country, emb_currency,
           emb_hour_of_day, W, b):
    rep_off = jnp.arange(N, dtype=jnp.int32) // (N // REP)
    hour_pad = jnp.pad(emb_hour_of_day, ((0, 0), (0, 16)))
    idx_flat = (merchant_id.reshape(N), category.reshape(N), mcc.reshape(N),
                country.reshape(N) + rep_off * 200,
                currency.reshape(N) + rep_off * 50,
                hour_of_day.reshape(N) + rep_off * 24)
    idxs = [a.reshape(IDX_ROWS, 128) for a in idx_flat]
    tabs = (emb_merchant_id, emb_category, emb_mcc,
            jnp.tile(emb_country, (REP, 1)), jnp.tile(emb_currency, (REP, 1)),
            jnp.tile(hour_pad, (REP, 1)))
    cat = _sc_gather_concat(idxs, tabs)
    w_t_pad = jnp.zeros((PROJ, PROJ), jnp.float32).at[:TOTAL].set(W.T)
    out = _tc_project(cat, w_t_pad, b.reshape(1, PROJ))
    return out.reshape(B, L, PROJ)


# speedup vs baseline: 22.3903x; 1.0762x over previous
import functools

import jax
import jax.numpy as jnp
from jax import lax
from jax.experimental import pallas as pl
from jax.experimental.pallas import tpu as pltpu
from jax.experimental.pallas import tpu_sc as plsc

B, L = 4096, 200
N = B * L
DIMS = (32, 16, 16, 16, 16, 32)
TOTAL = 112
PROJ = 128

SLICES = 2
NS_POS = N // SLICES
NC, NS = 2, 16
NW = NC * NS
PER_W = NS_POS // NW
CHUNK = 256
KROW = CHUNK // 128
STEPS = PER_W // CHUNK
IDX_ROWS = N // 128
REP = 128

_PACK = ((0, 0, 0), (0, 16, 16), (1, 0, 32), (2, 0, 48), (3, 0, 64),
         (4, 0, 80), (5, 0, 96), (5, 16, 112))


def _sc_gather_concat(idxs, tabs, s):
    mesh = plsc.VectorSubcoreMesh(core_axis_name="c", subcore_axis_name="s")
    out_type = jax.ShapeDtypeStruct((NS_POS, PROJ), jnp.float32)
    scratch = []
    for _b in range(2):
        for _f in range(6):
            scratch.append(pltpu.VMEM((KROW, 128), jnp.int32))
        for d in DIMS:
            scratch.append(pltpu.VMEM((CHUNK, d), jnp.float32))
    scratch.append(pltpu.VMEM((CHUNK, PROJ), jnp.float32))
    for _s in range(5):
        scratch.append(pltpu.SemaphoreType.DMA)

    @functools.partial(pl.kernel, out_type=out_type, mesh=mesh,
                       scratch_types=scratch,
                       compiler_params=pltpu.CompilerParams(
                           use_tc_tiling_on_sc=False))
    def body(i0, i1, i2, i3, i4, i5, t0, t1, t2, t3, t4, t5, out,
             a0, a1, a2, a3, a4, a5, ra0, ra1, ra2, ra3, ra4, ra5,
             b0, b1, b2, b3, b4, b5, rb0, rb1, rb2, rb3, rb4, rb5,
             cat, semi0, semi1, semg0, semg1, semw):
        idx_refs = (i0, i1, i2, i3, i4, i5)
        tab_refs = (t0, t1, t2, t3, t4, t5)
        ibufs = ((a0, a1, a2, a3, a4, a5), (b0, b1, b2, b3, b4, b5))
        rbufs = ((ra0, ra1, ra2, ra3, ra4, ra5),
                 (rb0, rb1, rb2, rb3, rb4, rb5))
        semi = (semi0, semi1)
        semg = (semg0, semg1)
        wid = lax.axis_index("s") * NC + lax.axis_index("c")
        row_base = s * (NS_POS // 128) + wid * (PER_W // 128)
        pos_base = wid * PER_W

        def fire_idx(g, sl):
            g = jnp.minimum(g, STEPS - 1)
            for f in range(6):
                pltpu.async_copy(
                    idx_refs[f].at[pl.ds(row_base + g * KROW, KROW)],
                    ibufs[sl][f], semi[sl])

        def drain_idx(sl):
            for f in range(6):
                pltpu.make_async_copy(
                    idx_refs[f].at[pl.ds(row_base, KROW)],
                    ibufs[sl][f], semi[sl]).wait()

        def fire_gathers(sl):
            for f in range(6):
                for j in range(KROW):
                    pltpu.async_copy(
                        tab_refs[f].at[ibufs[sl][f].at[j]],
                        rbufs[sl][f].at[pl.ds(j * 128, 128)], semg[sl])

        def wait_gathers(sl):
            for f in range(6):
                for j in range(KROW):
                    pltpu.make_async_copy(
                        tab_refs[f].at[ibufs[sl][f].at[j]],
                        rbufs[sl][f].at[pl.ds(j * 128, 128)],
                        semg[sl]).wait()

        def repack(sl):
            def row(r, carry):
                for f, soff, doff in _PACK:
                    cat[r, pl.ds(doff, 16)] = rbufs[sl][f][r, pl.ds(soff, 16)]
                return carry
            lax.fori_loop(0, CHUNK, row, 0)

        def fire_write(g):
            pltpu.async_copy(
                cat, out.at[pl.ds(pos_base + g * CHUNK, CHUNK)], semw)

        def drain_write():
            pltpu.make_async_copy(
                cat, out.at[pl.ds(pos_base, CHUNK)], semw).wait()

        def half(c, sl, fire_next, first_w):
            other = 1 - sl
            if fire_next:
                drain_idx(other)
                fire_gathers(other)
            wait_gathers(sl)
            fire_idx(c + 2, sl)
            if not first_w:
                drain_write()
            repack(sl)
            fire_write(c)

        fire_idx(jnp.int32(0), 0)
        fire_idx(jnp.int32(1), 1)
        drain_idx(0)
        fire_gathers(0)
        half(jnp.int32(0), 0, True, True)
        half(jnp.int32(1), 1, True, False)

        def step(k, carry):
            half(2 * k, 0, True, False)
            half(2 * k + 1, 1, True, False)
            return carry

        lax.fori_loop(1, STEPS // 2 - 1, step, 0)

        half(jnp.int32(STEPS - 2), 0, True, False)
        half(jnp.int32(STEPS - 1), 1, False, False)

        drain_write()
        drain_idx(0)
        drain_idx(1)

    return body(*idxs, *tabs)


def _tc_project(cat_s, w_t_pad, b2, s, prev):
    BLK = 4096
    grid = (NS_POS // BLK,)
    off = s * (NS_POS // BLK)

    def body(*refs):
        x, wt, bb, out = refs[0], refs[1], refs[2], refs[-1]
        out[...] = jnp.dot(x[...], wt[...],
                           preferred_element_type=jnp.float32) + bb[...]

    in_specs = [pl.BlockSpec((BLK, PROJ), lambda i: (i, 0)),
                pl.BlockSpec((PROJ, PROJ), lambda i: (0, 0)),
                pl.BlockSpec((1, PROJ), lambda i: (0, 0))]
    args = [cat_s, w_t_pad, b2]
    aliases = {}
    if prev is not None:
        in_specs.append(pl.BlockSpec(memory_space=pl.ANY))
        args.append(prev)
        aliases = {3: 0}
    return pl.pallas_call(
        body, grid=grid, in_specs=in_specs,
        out_specs=pl.BlockSpec((BLK, PROJ), lambda i: (i + off, 0)),
        out_shape=jax.ShapeDtypeStruct((N, PROJ), jnp.float32),
        input_output_aliases=aliases,
    )(*args)


def kernel(merchant_id, category, mcc, country, currency, hour_of_day,
           emb_merchant_id, emb_category, emb_mcc, emb_country, emb_currency,
           emb_hour_of_day, W, b):
    rep_off = jnp.arange(N, dtype=jnp.int32) // (N // REP)
    hour_pad = jnp.pad(emb_hour_of_day, ((0, 0), (0, 16)))
    idx_flat = (merchant_id.reshape(N), category.reshape(N), mcc.reshape(N),
                country.reshape(N) + rep_off * 200,
                currency.reshape(N) + rep_off * 50,
                hour_of_day.reshape(N) + rep_off * 24)
    idxs = [a.reshape(IDX_ROWS, 128) for a in idx_flat]
    tabs = (emb_merchant_id, emb_category, emb_mcc,
            jnp.tile(emb_country, (REP, 1)), jnp.tile(emb_currency, (REP, 1)),
            jnp.tile(hour_pad, (REP, 1)))
    w_t_pad = jnp.zeros((PROJ, PROJ), jnp.float32).at[:TOTAL].set(W.T)
    b2 = b.reshape(1, PROJ)

    cats = [_sc_gather_concat(idxs, tabs, s) for s in range(SLICES)]
    out = None
    for s in range(SLICES):
        out = _tc_project(cats[s], w_t_pad, b2, s, out)
    return out.reshape(B, L, PROJ)


# speedup vs baseline: 27.0653x; 1.2088x over previous
import functools

import jax
import jax.numpy as jnp
from jax import lax
from jax.experimental import pallas as pl
from jax.experimental.pallas import tpu as pltpu
from jax.experimental.pallas import tpu_sc as plsc

B, L = 4096, 200
N = B * L
DIMS = (32, 16, 16, 16, 16, 32)
TOTAL = 112
PROJ = 128

SLICES = 4
NS_POS = N // SLICES
NC, NS = 2, 16
NW = NC * NS
PER_W = NS_POS // NW
CHUNK = 128
KROW = CHUNK // 128
STEPS = PER_W // CHUNK
IDX_ROWS = N // 128
REP = 128

_PACK = ((0, 0, 0), (0, 16, 16), (1, 0, 32), (2, 0, 48), (3, 0, 64),
         (4, 0, 80), (5, 0, 96), (5, 16, 112))


def _sc_gather_concat(idxs, tabs, s):
    mesh = plsc.VectorSubcoreMesh(core_axis_name="c", subcore_axis_name="s")
    out_type = jax.ShapeDtypeStruct((NS_POS, PROJ), jnp.float32)
    scratch = []
    for _b in range(2):
        for _f in range(6):
            scratch.append(pltpu.VMEM((KROW, 128), jnp.int32))
        for d in DIMS:
            scratch.append(pltpu.VMEM((CHUNK, d), jnp.float32))
    scratch.append(pltpu.VMEM((CHUNK, PROJ), jnp.float32))
    for _s in range(5):
        scratch.append(pltpu.SemaphoreType.DMA)

    @functools.partial(pl.kernel, out_type=out_type, mesh=mesh,
                       scratch_types=scratch,
                       compiler_params=pltpu.CompilerParams(
                           use_tc_tiling_on_sc=False))
    def body(i0, i1, i2, i3, i4, i5, t0, t1, t2, t3, t4, t5, out,
             a0, a1, a2, a3, a4, a5, ra0, ra1, ra2, ra3, ra4, ra5,
             b0, b1, b2, b3, b4, b5, rb0, rb1, rb2, rb3, rb4, rb5,
             cat, semi0, semi1, semg0, semg1, semw):
        idx_refs = (i0, i1, i2, i3, i4, i5)
        tab_refs = (t0, t1, t2, t3, t4, t5)
        ibufs = ((a0, a1, a2, a3, a4, a5), (b0, b1, b2, b3, b4, b5))
        rbufs = ((ra0, ra1, ra2, ra3, ra4, ra5),
                 (rb0, rb1, rb2, rb3, rb4, rb5))
        semi = (semi0, semi1)
        semg = (semg0, semg1)
        wid = lax.axis_index("s") * NC + lax.axis_index("c")
        row_base = s * (NS_POS // 128) + wid * (PER_W // 128)
        pos_base = wid * PER_W

        def fire_idx(g, sl):
            g = jnp.minimum(g, STEPS - 1)
            for f in range(6):
                pltpu.async_copy(
                    idx_refs[f].at[pl.ds(row_base + g * KROW, KROW)],
                    ibufs[sl][f], semi[sl])

        def drain_idx(sl):
            for f in range(6):
                pltpu.make_async_copy(
                    idx_refs[f].at[pl.ds(row_base, KROW)],
                    ibufs[sl][f], semi[sl]).wait()

        def fire_gathers(sl):
            for f in range(6):
                for j in range(KROW):
                    pltpu.async_copy(
                        tab_refs[f].at[ibufs[sl][f].at[j]],
                        rbufs[sl][f].at[pl.ds(j * 128, 128)], semg[sl])

        def wait_gathers(sl):
            for f in range(6):
                for j in range(KROW):
                    pltpu.make_async_copy(
                        tab_refs[f].at[ibufs[sl][f].at[j]],
                        rbufs[sl][f].at[pl.ds(j * 128, 128)],
                        semg[sl]).wait()

        def repack(sl):
            @plsc.parallel_loop(0, CHUNK, unroll=4)
            def row(r):
                for f, soff, doff in _PACK:
                    cat[r, pl.ds(doff, 16)] = rbufs[sl][f][r, pl.ds(soff, 16)]

        def fire_write(g):
            pltpu.async_copy(
                cat, out.at[pl.ds(pos_base + g * CHUNK, CHUNK)], semw)

        def drain_write():
            pltpu.make_async_copy(
                cat, out.at[pl.ds(pos_base, CHUNK)], semw).wait()

        def half(c, sl, fire_next, first_w):
            other = 1 - sl
            if fire_next:
                drain_idx(other)
                fire_gathers(other)
            wait_gathers(sl)
            fire_idx(c + 2, sl)
            if not first_w:
                drain_write()
            repack(sl)
            fire_write(c)

        fire_idx(jnp.int32(0), 0)
        fire_idx(jnp.int32(1), 1)
        drain_idx(0)
        fire_gathers(0)
        half(jnp.int32(0), 0, True, True)
        half(jnp.int32(1), 1, True, False)

        def step(k, carry):
            half(2 * k, 0, True, False)
            half(2 * k + 1, 1, True, False)
            return carry

        lax.fori_loop(1, STEPS // 2 - 1, step, 0)

        half(jnp.int32(STEPS - 2), 0, True, False)
        half(jnp.int32(STEPS - 1), 1, False, False)

        drain_write()
        drain_idx(0)
        drain_idx(1)

    return body(*idxs, *tabs)


def _tc_project(cat_s, w_t_pad, b2, s, prev):
    BLK = 4096
    grid = (NS_POS // BLK,)
    off = s * (NS_POS // BLK)

    def body(*refs):
        x, wt, bb, out = refs[0], refs[1], refs[2], refs[-1]
        out[...] = jnp.dot(x[...], wt[...],
                           preferred_element_type=jnp.float32) + bb[...]

    in_specs = [pl.BlockSpec((BLK, PROJ), lambda i: (i, 0)),
                pl.BlockSpec((PROJ, PROJ), lambda i: (0, 0)),
                pl.BlockSpec((1, PROJ), lambda i: (0, 0))]
    args = [cat_s, w_t_pad, b2]
    aliases = {}
    if prev is not None:
        in_specs.append(pl.BlockSpec(memory_space=pl.ANY))
        args.append(prev)
        aliases = {3: 0}
    return pl.pallas_call(
        body, grid=grid, in_specs=in_specs,
        out_specs=pl.BlockSpec((BLK, PROJ), lambda i: (i + off, 0)),
        out_shape=jax.ShapeDtypeStruct((N, PROJ), jnp.float32),
        input_output_aliases=aliases,
    )(*args)


def kernel(merchant_id, category, mcc, country, currency, hour_of_day,
           emb_merchant_id, emb_category, emb_mcc, emb_country, emb_currency,
           emb_hour_of_day, W, b):
    rep_off = jnp.arange(N, dtype=jnp.int32) // (N // REP)
    hour_pad = jnp.pad(emb_hour_of_day, ((0, 0), (0, 16)))
    idx_flat = (merchant_id.reshape(N), category.reshape(N), mcc.reshape(N),
                country.reshape(N) + rep_off * 200,
                currency.reshape(N) + rep_off * 50,
                hour_of_day.reshape(N) + rep_off * 24)
    idxs = [a.reshape(IDX_ROWS, 128) for a in idx_flat]
    tabs = (emb_merchant_id, emb_category, emb_mcc,
            jnp.tile(emb_country, (REP, 1)), jnp.tile(emb_currency, (REP, 1)),
            jnp.tile(hour_pad, (REP, 1)))
    w_t_pad = jnp.zeros((PROJ, PROJ), jnp.float32).at[:TOTAL].set(W.T)
    b2 = b.reshape(1, PROJ)

    cats = [_sc_gather_concat(idxs, tabs, s) for s in range(SLICES)]
    out = None
    for s in range(SLICES):
        out = _tc_project(cats[s], w_t_pad, b2, s, out)
    return out.reshape(B, L, PROJ)
